# Initial kernel scaffold; baseline (speedup 1.0000x reference)
import jax
import jax.numpy as jnp
from jax.experimental import pallas as pl


def kernel(x, masks, W0, attn_l0, attn_r0, bias0):
    raise NotImplementedError("write your pallas kernel here")


# speedup vs baseline: 499.3651x; 499.3651x over previous
import functools
import jax
import jax.numpy as jnp
from jax.experimental import pallas as pl
from jax.experimental.pallas import tpu as pltpu

B, S, C, T = 4, 8, 128, 32
HEADS, HIDDEN = 4, 32
G = B * S
Q = T * T
HC = HEADS * HIDDEN

NEG = -1e30
OFFSETS = [(di, dj) for di in range(-2, 3) for dj in range(-2, 3)]


def _roll(a, shift):
    if shift % a.shape[-1] == 0:
        return a
    return jnp.roll(a, shift, axis=-1)


def _gat_body(x_ref, maskq_ref, w_ref, bias_ref, out_ref):
    xg = x_ref[0]
    zz = jnp.dot(w_ref[:], xg, preferred_element_type=jnp.float32)
    z = zz[:HC]
    el = zz[HC:HC + HEADS]
    er = zz[HC + HEADS:HC + 2 * HEADS]

    node_mask = maskq_ref[0, 0] != 0
    el_m = jnp.where(node_mask[None, :], el, NEG)

    q_iota = jax.lax.broadcasted_iota(jnp.int32, (1, Q), 1)
    qi = q_iota // T
    qj = q_iota % T

    def edge_logit(di, dj):
        off = di * T + dj
        els = _roll(el_m, -off)
        e = els + er
        e = jnp.where(e > 0, e, 0.2 * e)
        inb = (qi + di >= 0) & (qi + di < T) & (qj + dj >= 0) & (qj + dj < T)
        return jnp.where(inb, e, NEG)

    m = jnp.full((HEADS, Q), NEG, dtype=jnp.float32)
    for (di, dj) in OFFSETS:
        m = jnp.maximum(m, edge_logit(di, dj))

    row_iota = jax.lax.broadcasted_iota(jnp.int32, (HC, HEADS), 0)
    col_iota = jax.lax.broadcasted_iota(jnp.int32, (HC, HEADS), 1)
    expand = (row_iota // HIDDEN == col_iota).astype(jnp.float32)

    denom = jnp.zeros((HEADS, Q), dtype=jnp.float32)
    acc = jnp.zeros((HC, Q), dtype=jnp.float32)
    for (di, dj) in OFFSETS:
        e = edge_logit(di, dj)
        w = jnp.where(e > -1e20, jnp.exp(e - m), 0.0)
        denom = denom + w
        w_exp = jnp.dot(expand, w, preferred_element_type=jnp.float32)
        off = di * T + dj
        acc = acc + w_exp * _roll(z, -off)

    denom = jnp.where(denom > 0, denom, 1.0)
    den_exp = jnp.dot(expand, denom, preferred_element_type=jnp.float32)
    out = acc / den_exp + bias_ref[:].reshape(HC, 1)
    out = jnp.where(out > 0, out, jnp.exp(jnp.minimum(out, 0.0)) - 1.0)
    out = jnp.where(node_mask[None, :], out, 0.0)
    out_ref[0] = out


def kernel(x, masks, W0, attn_l0, attn_r0, bias0):
    xg = x.reshape(G, C, Q)
    maskq = masks.reshape(G, 1, Q)

    eye = (jnp.arange(HEADS)[:, None] == (jnp.arange(HC) // HIDDEN)[None, :])
    Al = eye.astype(jnp.float32) * jnp.tile(attn_l0, (1, HEADS))
    Ar = eye.astype(jnp.float32) * jnp.tile(attn_r0, (1, HEADS))
    wbig = jnp.concatenate([W0, Al @ W0, Ar @ W0], axis=0)

    out = pl.pallas_call(
        _gat_body,
        grid=(G,),
        in_specs=[
            pl.BlockSpec((1, C, Q), lambda g: (g, 0, 0)),
            pl.BlockSpec((1, 1, Q), lambda g: (0, 0, 0)),
            pl.BlockSpec((HC + 2 * HEADS, C), lambda g: (0, 0)),
            pl.BlockSpec((HC,), lambda g: (0,)),
        ],
        out_specs=pl.BlockSpec((1, HC, Q), lambda g: (g, 0, 0)),
        out_shape=jax.ShapeDtypeStruct((G, HC, Q), jnp.float32),
    )(xg, maskq, wbig, bias0)

    return out.reshape(x.shape)
